# Initial kernel scaffold; baseline (speedup 1.0000x reference)
import jax
import jax.numpy as jnp
from jax.experimental import pallas as pl


def kernel(alpha, eps, dec_token, w_enc, w1, b1, bn_scale, bn_shift, w2, b2, w_out, b_out, bond_emb_0, bond_emb_1, bond_emb_2, x, edge_index, edge_attr, masked_node_indices):
    raise NotImplementedError("write your pallas kernel here")


# speedup vs baseline: 1.2984x; 1.2984x over previous
import functools

import jax
import jax.numpy as jnp
from jax import lax
from jax.experimental import pallas as pl
from jax.experimental.pallas import tpu as pltpu

_BF16 = jnp.bfloat16
_F32 = jnp.float32


def _round_up(x, m):
    return (x + m - 1) // m * m


def _encode_kernel(x_ref, mask_ref, alpha_ref, w_enc_ref, dec_tok_ref, h_ref):
    x = x_ref[...]
    a = jnp.where(x >= 0.0, x, alpha_ref[...] * x)
    h = jnp.dot(a.astype(_BF16), w_enc_ref[...], preferred_element_type=_F32)
    h = jnp.where(mask_ref[...] > 0.0, dec_tok_ref[...], h)
    h_ref[...] = h.astype(_BF16)


def _message_kernel(h_ref, src_ref, dst_ref, ea0_ref, ea1_ref, ea2_ref,
                    tab_ref, agg_ref, *, n_chunk, s_chunk):
    @pl.when(pl.program_id(1) == 0)
    def _init():
        agg_ref[...] = jnp.zeros(agg_ref.shape, agg_ref.dtype)

    te = src_ref.shape[0]
    n_pad, h_dim = h_ref.shape

    src = src_ref[...]
    x_j = jnp.zeros((te, h_dim), _F32)
    for kb in range(n_pad // n_chunk):
        ids = lax.broadcasted_iota(jnp.int32, (te, n_chunk), 1) + (kb * n_chunk)
        oh = (ids == src).astype(_BF16)
        x_j = x_j + jnp.dot(oh, h_ref[kb * n_chunk:(kb + 1) * n_chunk, :],
                            preferred_element_type=_F32)

    nv = tab_ref.shape[0]
    it = lax.broadcasted_iota(jnp.int32, (te, nv), 1)
    ohe = ((it == ea0_ref[...]) | (it == ea1_ref[...])
           | (it == ea2_ref[...])).astype(_F32)
    emb = jnp.dot(ohe, tab_ref[...], preferred_element_type=_F32)

    msg = jnp.maximum(x_j + emb, 0.0).astype(_BF16)

    dst = dst_ref[...]
    for rb in range(n_pad // s_chunk):
        ids = lax.broadcasted_iota(jnp.int32, (s_chunk, te), 0) + (rb * s_chunk)
        ohd = (ids == dst).astype(_BF16)
        agg_ref[rb * s_chunk:(rb + 1) * s_chunk, :] += jnp.dot(
            ohd, msg, preferred_element_type=_F32)


def _mlp_kernel(h_ref, agg0_ref, agg1_ref, ope_ref, w1_ref, b1_ref,
                w23_ref, b23_ref, out_ref):
    h = h_ref[...].astype(_F32)
    pre = ope_ref[...] * h + (agg0_ref[...] + agg1_ref[...])
    hid = jnp.maximum(
        jnp.dot(pre.astype(_BF16), w1_ref[...], preferred_element_type=_F32)
        + b1_ref[...], 0.0)
    out_ref[...] = (jnp.dot(hid.astype(_BF16), w23_ref[...],
                            preferred_element_type=_F32) + b23_ref[...])


def kernel(alpha, eps, dec_token, w_enc, w1, b1, bn_scale, bn_shift, w2, b2,
           w_out, b_out, bond_emb_0, bond_emb_1, bond_emb_2,
           x, edge_index, edge_attr, masked_node_indices):
    N, H = x.shape
    E = edge_index.shape[1]
    out_dim = w_out.shape[1]

    H_pad = _round_up(H, 128)
    H2_pad = _round_up(2 * H, 128)
    O_pad = _round_up(out_dim, 128)

    TN = 512
    TE = 512
    N_CHUNK = 2048
    S_CHUNK = 512

    N_pad = _round_up(N, N_CHUNK)
    TE = _round_up(min(TE, _round_up(E, 128)), 128)
    E_pad = _round_up(E, 2 * TE)

    def pad2(a, r, c):
        a = a.astype(_F32)
        return jnp.zeros((r, c), _F32).at[:a.shape[0], :a.shape[1]].set(a)

    x_p = pad2(x, N_pad, H_pad)
    mask = jnp.zeros((N_pad, 1), _F32).at[masked_node_indices, 0].set(1.0)
    w_enc_p = pad2(w_enc, H_pad, H_pad).astype(_BF16)
    dec_tok_p = pad2(dec_token, 1, H_pad)

    w1f = w1 * bn_scale
    b1f = b1 * bn_scale + bn_shift
    w23 = w2 @ w_out
    b23 = b2 @ w_out + b_out
    w1f_p = pad2(w1f, H_pad, H2_pad).astype(_BF16)
    b1f_p = pad2(b1f, 1, H2_pad)
    w23_p = pad2(w23, H2_pad, O_pad).astype(_BF16)
    b23_p = pad2(b23, 1, O_pad)

    v0 = bond_emb_0.shape[0]
    v1 = bond_emb_1.shape[0]
    v2 = bond_emb_2.shape[0]
    s1 = _round_up(v0, 8)
    s2 = s1 + _round_up(v1, 8)
    nv = s2 + _round_up(v2, 8)
    tab = jnp.zeros((nv, H_pad), _F32)
    tab = tab.at[:v0, :H].set(bond_emb_0.astype(_F32))
    tab = tab.at[s1:s1 + v1, :H].set(bond_emb_1.astype(_F32))
    tab = tab.at[s2:s2 + v2, :H].set(bond_emb_2.astype(_F32))

    ea = edge_attr.astype(jnp.int32)
    ea0_p = jnp.full((E_pad, 1), -1, jnp.int32).at[:E, 0].set(ea[:, 0])
    ea1_p = jnp.full((E_pad, 1), -1, jnp.int32).at[:E, 0].set(ea[:, 1] + s1)
    ea2_p = jnp.full((E_pad, 1), -1, jnp.int32).at[:E, 0].set(ea[:, 2] + s2)

    src_p = (jnp.zeros((E_pad, 1), jnp.int32)
             .at[:E, 0].set(edge_index[0].astype(jnp.int32)))
    dst_p = (jnp.full((1, E_pad), -1, jnp.int32)
             .at[0, :E].set(edge_index[1].astype(jnp.int32)))

    alpha_row = jnp.broadcast_to(alpha.astype(_F32), (1, H_pad))
    ope_row = jnp.broadcast_to((1.0 + eps).astype(_F32), (1, H_pad))

    n_node_tiles = N_pad // TN
    n_edge_tiles = E_pad // TE
    tiles_per_core = n_edge_tiles // 2

    h_bf = pl.pallas_call(
        _encode_kernel,
        out_shape=jax.ShapeDtypeStruct((N_pad, H_pad), _BF16),
        grid=(n_node_tiles,),
        in_specs=[
            pl.BlockSpec((TN, H_pad), lambda i: (i, 0)),
            pl.BlockSpec((TN, 1), lambda i: (i, 0)),
            pl.BlockSpec((1, H_pad), lambda i: (0, 0)),
            pl.BlockSpec((H_pad, H_pad), lambda i: (0, 0)),
            pl.BlockSpec((1, H_pad), lambda i: (0, 0)),
        ],
        out_specs=pl.BlockSpec((TN, H_pad), lambda i: (i, 0)),
        compiler_params=pltpu.CompilerParams(
            dimension_semantics=("parallel",), vmem_limit_bytes=32 << 20),
    )(x_p, mask, alpha_row, w_enc_p, dec_tok_p)

    agg2 = pl.pallas_call(
        functools.partial(_message_kernel, n_chunk=N_CHUNK, s_chunk=S_CHUNK),
        out_shape=jax.ShapeDtypeStruct((2 * N_pad, H_pad), _F32),
        grid=(2, tiles_per_core),
        in_specs=[
            pl.BlockSpec((N_pad, H_pad), lambda c, t: (0, 0)),
            pl.BlockSpec((TE, 1), lambda c, t, _n=tiles_per_core: (c * _n + t, 0)),
            pl.BlockSpec((1, TE), lambda c, t, _n=tiles_per_core: (0, c * _n + t)),
            pl.BlockSpec((TE, 1), lambda c, t, _n=tiles_per_core: (c * _n + t, 0)),
            pl.BlockSpec((TE, 1), lambda c, t, _n=tiles_per_core: (c * _n + t, 0)),
            pl.BlockSpec((TE, 1), lambda c, t, _n=tiles_per_core: (c * _n + t, 0)),
            pl.BlockSpec((nv, H_pad), lambda c, t: (0, 0)),
        ],
        out_specs=pl.BlockSpec((N_pad, H_pad), lambda c, t: (c, 0)),
        compiler_params=pltpu.CompilerParams(
            dimension_semantics=("parallel", "arbitrary"),
            vmem_limit_bytes=60 << 20),
    )(h_bf, src_p, dst_p, ea0_p, ea1_p, ea2_p, tab)

    n_half = N_pad // TN
    out_p = pl.pallas_call(
        _mlp_kernel,
        out_shape=jax.ShapeDtypeStruct((N_pad, O_pad), _F32),
        grid=(n_node_tiles,),
        in_specs=[
            pl.BlockSpec((TN, H_pad), lambda i: (i, 0)),
            pl.BlockSpec((TN, H_pad), lambda i: (i, 0)),
            pl.BlockSpec((TN, H_pad), lambda i, _n=n_half: (_n + i, 0)),
            pl.BlockSpec((1, H_pad), lambda i: (0, 0)),
            pl.BlockSpec((H_pad, H2_pad), lambda i: (0, 0)),
            pl.BlockSpec((1, H2_pad), lambda i: (0, 0)),
            pl.BlockSpec((H2_pad, O_pad), lambda i: (0, 0)),
            pl.BlockSpec((1, O_pad), lambda i: (0, 0)),
        ],
        out_specs=pl.BlockSpec((TN, O_pad), lambda i: (i, 0)),
        compiler_params=pltpu.CompilerParams(
            dimension_semantics=("parallel",), vmem_limit_bytes=32 << 20),
    )(h_bf, agg2, agg2, ope_row, w1f_p, b1f_p, w23_p, b23_p)

    return out_p[:N, :out_dim]


# speedup vs baseline: 7.7889x; 5.9987x over previous
import functools

import jax
import jax.numpy as jnp
from jax import lax
from jax.experimental import pallas as pl
from jax.experimental.pallas import tpu as pltpu

_BF16 = jnp.bfloat16
_F32 = jnp.float32


def _round_up(x, m):
    return (x + m - 1) // m * m


def _encode_kernel(x_ref, mask_ref, alpha_ref, w_enc_ref, dec_tok_ref, h_ref):
    x = x_ref[...]
    a = jnp.where(x >= 0.0, x, alpha_ref[...] * x)
    h = jnp.dot(a.astype(_BF16), w_enc_ref[...], preferred_element_type=_F32)
    h_ref[...] = jnp.where(mask_ref[...] > 0.0, dec_tok_ref[...], h)


def _message_kernel(psd_ref, c_ref, h_ref, tab_ref, agg_ref, *,
                    n_edges, unroll, bits):
    agg_ref[...] = jnp.zeros(agg_ref.shape, agg_ref.dtype)
    mask = (1 << bits) - 1

    def body(i, carry):
        base = i * unroll
        for j in range(unroll):
            p = psd_ref[base + j]
            cc = c_ref[base + j]
            s = p >> bits
            d = p & mask
            row = h_ref[pl.ds(s, 1), :, :] + tab_ref[pl.ds(cc, 1), :, :]
            m = jnp.maximum(row, 0.0)
            agg_ref[pl.ds(d, 1), :, :] = agg_ref[pl.ds(d, 1), :, :] + m
        return carry

    lax.fori_loop(0, n_edges // unroll, body, 0)


def _mlp_kernel(h_ref, agg_ref, ope_ref, w1_ref, b1_ref, w23_ref, b23_ref,
                out_ref):
    pre = ope_ref[...] * h_ref[...] + agg_ref[...]
    hid = jnp.maximum(
        jnp.dot(pre.astype(_BF16), w1_ref[...], preferred_element_type=_F32)
        + b1_ref[...], 0.0)
    out_ref[...] = (jnp.dot(hid.astype(_BF16), w23_ref[...],
                            preferred_element_type=_F32) + b23_ref[...])


def kernel(alpha, eps, dec_token, w_enc, w1, b1, bn_scale, bn_shift, w2, b2,
           w_out, b_out, bond_emb_0, bond_emb_1, bond_emb_2,
           x, edge_index, edge_attr, masked_node_indices):
    N, H = x.shape
    E = edge_index.shape[1]
    out_dim = w_out.shape[1]

    H_pad = _round_up(H, 128)
    H2_pad = _round_up(2 * H, 128)
    O_pad = _round_up(out_dim, 128)

    TN = 512
    UNROLL = 8

    N_pad = _round_up(N, TN)
    E_pad = _round_up(E, UNROLL)
    bits = int(N_pad - 1).bit_length()

    def pad2(a, r, c):
        a = a.astype(_F32)
        return jnp.zeros((r, c), _F32).at[:a.shape[0], :a.shape[1]].set(a)

    x_p = pad2(x, N_pad, H_pad)
    mask = jnp.zeros((N_pad, 1), _F32).at[masked_node_indices, 0].set(1.0)
    w_enc_p = pad2(w_enc, H_pad, H_pad).astype(_BF16)
    dec_tok_p = pad2(dec_token, 1, H_pad)

    w1f = w1 * bn_scale
    b1f = b1 * bn_scale + bn_shift
    w23 = w2 @ w_out
    b23 = b2 @ w_out + b_out
    w1f_p = pad2(w1f, H_pad, H2_pad).astype(_BF16)
    b1f_p = pad2(b1f, 1, H2_pad)
    w23_p = pad2(w23, H2_pad, O_pad).astype(_BF16)
    b23_p = pad2(b23, 1, O_pad)

    v0 = bond_emb_0.shape[0]
    v1 = bond_emb_1.shape[0]
    v2 = bond_emb_2.shape[0]
    nv = v0 * v1 * v2
    nv_pad = _round_up(nv + 1, 8)
    tabc = (bond_emb_0.astype(_F32)[:, None, None, :]
            + bond_emb_1.astype(_F32)[None, :, None, :]
            + bond_emb_2.astype(_F32)[None, None, :, :]).reshape(nv, H)
    tabc_p = pad2(tabc, nv_pad, H_pad).reshape(nv_pad, 1, H_pad)

    ea = edge_attr.astype(jnp.int32)
    c_ids = (ea[:, 0] * (v1 * v2) + ea[:, 1] * v2 + ea[:, 2]).astype(jnp.int32)
    src = edge_index[0].astype(jnp.int32)
    dst = edge_index[1].astype(jnp.int32)
    n_rows = N_pad + 8 if E_pad > E else N_pad
    psd = jnp.full((E_pad,), jnp.int32(N_pad) if E_pad > E else 0, jnp.int32)
    psd = psd.at[:E].set((src << bits) | dst)
    c_p = jnp.full((E_pad,), nv, jnp.int32).at[:E].set(c_ids)

    alpha_row = jnp.broadcast_to(alpha.astype(_F32), (1, H_pad))
    ope_row = jnp.broadcast_to((1.0 + eps).astype(_F32), (1, H_pad))

    n_node_tiles = N_pad // TN

    h = pl.pallas_call(
        _encode_kernel,
        out_shape=jax.ShapeDtypeStruct((N_pad, H_pad), _F32),
        grid=(n_node_tiles,),
        in_specs=[
            pl.BlockSpec((TN, H_pad), lambda i: (i, 0)),
            pl.BlockSpec((TN, 1), lambda i: (i, 0)),
            pl.BlockSpec((1, H_pad), lambda i: (0, 0)),
            pl.BlockSpec((H_pad, H_pad), lambda i: (0, 0)),
            pl.BlockSpec((1, H_pad), lambda i: (0, 0)),
        ],
        out_specs=pl.BlockSpec((TN, H_pad), lambda i: (i, 0)),
        compiler_params=pltpu.CompilerParams(
            dimension_semantics=("parallel",), vmem_limit_bytes=32 << 20),
    )(x_p, mask, alpha_row, w_enc_p, dec_tok_p)

    h3 = h.reshape(N_pad, 1, H_pad)
    agg3 = pl.pallas_call(
        functools.partial(_message_kernel, n_edges=E_pad, unroll=UNROLL,
                          bits=bits),
        out_shape=jax.ShapeDtypeStruct((n_rows, 1, H_pad), _F32),
        grid=(1,),
        in_specs=[
            pl.BlockSpec(memory_space=pltpu.SMEM),
            pl.BlockSpec(memory_space=pltpu.SMEM),
            pl.BlockSpec((N_pad, 1, H_pad), lambda i: (0, 0, 0)),
            pl.BlockSpec((nv_pad, 1, H_pad), lambda i: (0, 0, 0)),
        ],
        out_specs=pl.BlockSpec((n_rows, 1, H_pad), lambda i: (0, 0, 0)),
        compiler_params=pltpu.CompilerParams(
            dimension_semantics=("arbitrary",), vmem_limit_bytes=48 << 20),
    )(psd, c_p, h3, tabc_p)

    agg = agg3[:N_pad].reshape(N_pad, H_pad) if n_rows != N_pad \
        else agg3.reshape(N_pad, H_pad)

    out_p = pl.pallas_call(
        _mlp_kernel,
        out_shape=jax.ShapeDtypeStruct((N_pad, O_pad), _F32),
        grid=(n_node_tiles,),
        in_specs=[
            pl.BlockSpec((TN, H_pad), lambda i: (i, 0)),
            pl.BlockSpec((TN, H_pad), lambda i: (i, 0)),
            pl.BlockSpec((1, H_pad), lambda i: (0, 0)),
            pl.BlockSpec((H_pad, H2_pad), lambda i: (0, 0)),
            pl.BlockSpec((1, H2_pad), lambda i: (0, 0)),
            pl.BlockSpec((H2_pad, O_pad), lambda i: (0, 0)),
            pl.BlockSpec((1, O_pad), lambda i: (0, 0)),
        ],
        out_specs=pl.BlockSpec((TN, O_pad), lambda i: (i, 0)),
        compiler_params=pltpu.CompilerParams(
            dimension_semantics=("parallel",), vmem_limit_bytes=32 << 20),
    )(h, agg, ope_row, w1f_p, b1f_p, w23_p, b23_p)

    return out_p[:N, :out_dim]


# speedup vs baseline: 7.9747x; 1.0239x over previous
import functools

import jax
import jax.numpy as jnp
from jax import lax
from jax.experimental import pallas as pl
from jax.experimental.pallas import tpu as pltpu

_BF16 = jnp.bfloat16
_F32 = jnp.float32


def _round_up(x, m):
    return (x + m - 1) // m * m


def _encode_kernel(x_ref, midx_ref, alpha_ref, w_enc_ref, dec_tok_ref, h_ref,
                   *, tn):
    x = x_ref[...]
    a = jnp.where(x >= 0.0, x, alpha_ref[...] * x)
    h = jnp.dot(a.astype(_BF16), w_enc_ref[...], preferred_element_type=_F32)
    row_ids = (lax.broadcasted_iota(jnp.int32, (tn, midx_ref.shape[1]), 0)
               + pl.program_id(0) * tn)
    is_masked = jnp.max((row_ids == midx_ref[...]).astype(_F32), axis=1,
                        keepdims=True)
    h_ref[...] = jnp.where(is_masked > 0.0, dec_tok_ref[...], h)


def _message_kernel(psd_ref, c_ref, h_ref, tab_ref, agg0_ref, agg1_ref, *,
                    n_edges, unroll, bits):
    agg0_ref[...] = jnp.zeros(agg0_ref.shape, agg0_ref.dtype)
    agg1_ref[...] = jnp.zeros(agg1_ref.shape, agg1_ref.dtype)
    mask = (1 << bits) - 1
    aggs = (agg0_ref, agg1_ref)

    def body(i, carry):
        base = i * unroll
        for j in range(unroll):
            p = psd_ref[base + j]
            cc = c_ref[base + j]
            s = p >> bits
            d = p & mask
            row = h_ref[pl.ds(s, 1), :, :] + tab_ref[pl.ds(cc, 1), :, :]
            m = jnp.maximum(row, 0.0)
            a = aggs[j % 2]
            a[pl.ds(d, 1), :, :] = a[pl.ds(d, 1), :, :] + m
        return carry

    lax.fori_loop(0, n_edges // unroll, body, 0)


def _mlp_kernel(h_ref, agg0_ref, agg1_ref, ope_ref, w1_ref, b1_ref, w23_ref,
                b23_ref, out_ref):
    pre = ope_ref[...] * h_ref[...] + (agg0_ref[...] + agg1_ref[...])
    hid = jnp.maximum(
        jnp.dot(pre.astype(_BF16), w1_ref[...], preferred_element_type=_F32)
        + b1_ref[...], 0.0)
    out_ref[...] = (jnp.dot(hid.astype(_BF16), w23_ref[...],
                            preferred_element_type=_F32) + b23_ref[...])


def kernel(alpha, eps, dec_token, w_enc, w1, b1, bn_scale, bn_shift, w2, b2,
           w_out, b_out, bond_emb_0, bond_emb_1, bond_emb_2,
           x, edge_index, edge_attr, masked_node_indices):
    N, H = x.shape
    E = edge_index.shape[1]
    out_dim = w_out.shape[1]
    n_masked = masked_node_indices.shape[0]

    H_pad = _round_up(H, 128)
    H2_pad = _round_up(2 * H, 128)
    O_pad = _round_up(out_dim, 128)
    M_pad = _round_up(n_masked, 128)

    TN = 512
    UNROLL = 8

    N_pad = _round_up(N, TN)
    E_pad = _round_up(E, UNROLL)
    bits = int(N_pad - 1).bit_length()

    def pad2(a, r, c):
        a = a.astype(_F32)
        if a.shape == (r, c):
            return a
        return jnp.zeros((r, c), _F32).at[:a.shape[0], :a.shape[1]].set(a)

    x_p = pad2(x, N_pad, H_pad)
    midx = jnp.full((1, M_pad), -1, jnp.int32).at[0, :n_masked].set(
        masked_node_indices.astype(jnp.int32))
    w_enc_p = pad2(w_enc, H_pad, H_pad).astype(_BF16)
    dec_tok_p = pad2(dec_token, 1, H_pad)

    w1f = w1 * bn_scale
    b1f = b1 * bn_scale + bn_shift
    w23 = w2 @ w_out
    b23 = b2 @ w_out + b_out
    w1f_p = pad2(w1f, H_pad, H2_pad).astype(_BF16)
    b1f_p = pad2(b1f, 1, H2_pad)
    w23_p = pad2(w23, H2_pad, O_pad).astype(_BF16)
    b23_p = pad2(b23, 1, O_pad)

    v0 = bond_emb_0.shape[0]
    v1 = bond_emb_1.shape[0]
    v2 = bond_emb_2.shape[0]
    nv = v0 * v1 * v2
    nv_pad = _round_up(nv + 1, 8)
    tabc = (bond_emb_0.astype(_F32)[:, None, None, :]
            + bond_emb_1.astype(_F32)[None, :, None, :]
            + bond_emb_2.astype(_F32)[None, None, :, :]).reshape(nv, H)
    tabc_p = pad2(tabc, nv_pad, H_pad).reshape(nv_pad, 1, H_pad)

    ea = edge_attr.astype(jnp.int32)
    c_ids = (ea[:, 0] * (v1 * v2) + ea[:, 1] * v2 + ea[:, 2]).astype(jnp.int32)
    src = edge_index[0].astype(jnp.int32)
    dst = edge_index[1].astype(jnp.int32)
    n_rows = N_pad + 8 if E_pad > E else N_pad
    psd = jnp.full((E_pad,), jnp.int32(N_pad) if E_pad > E else 0, jnp.int32)
    psd = psd.at[:E].set((src << bits) | dst)
    c_p = jnp.full((E_pad,), nv, jnp.int32).at[:E].set(c_ids)

    alpha_row = jnp.broadcast_to(alpha.astype(_F32), (1, H_pad))
    ope_row = jnp.broadcast_to((1.0 + eps).astype(_F32), (1, H_pad))

    n_node_tiles = N_pad // TN

    h = pl.pallas_call(
        functools.partial(_encode_kernel, tn=TN),
        out_shape=jax.ShapeDtypeStruct((N_pad, H_pad), _F32),
        grid=(n_node_tiles,),
        in_specs=[
            pl.BlockSpec((TN, H_pad), lambda i: (i, 0)),
            pl.BlockSpec((1, M_pad), lambda i: (0, 0)),
            pl.BlockSpec((1, H_pad), lambda i: (0, 0)),
            pl.BlockSpec((H_pad, H_pad), lambda i: (0, 0)),
            pl.BlockSpec((1, H_pad), lambda i: (0, 0)),
        ],
        out_specs=pl.BlockSpec((TN, H_pad), lambda i: (i, 0)),
        compiler_params=pltpu.CompilerParams(
            dimension_semantics=("parallel",), vmem_limit_bytes=32 << 20),
    )(x_p, midx, alpha_row, w_enc_p, dec_tok_p)

    h3 = h.reshape(N_pad, 1, H_pad)
    agg3a, agg3b = pl.pallas_call(
        functools.partial(_message_kernel, n_edges=E_pad, unroll=UNROLL,
                          bits=bits),
        out_shape=(jax.ShapeDtypeStruct((n_rows, 1, H_pad), _F32),
                   jax.ShapeDtypeStruct((n_rows, 1, H_pad), _F32)),
        grid=(1,),
        in_specs=[
            pl.BlockSpec(memory_space=pltpu.SMEM),
            pl.BlockSpec(memory_space=pltpu.SMEM),
            pl.BlockSpec((N_pad, 1, H_pad), lambda i: (0, 0, 0)),
            pl.BlockSpec((nv_pad, 1, H_pad), lambda i: (0, 0, 0)),
        ],
        out_specs=(pl.BlockSpec((n_rows, 1, H_pad), lambda i: (0, 0, 0)),
                   pl.BlockSpec((n_rows, 1, H_pad), lambda i: (0, 0, 0))),
        compiler_params=pltpu.CompilerParams(
            dimension_semantics=("arbitrary",), vmem_limit_bytes=56 << 20),
    )(psd, c_p, h3, tabc_p)

    def flat(a):
        a = a[:N_pad] if n_rows != N_pad else a
        return a.reshape(N_pad, H_pad)

    agg_a, agg_b = flat(agg3a), flat(agg3b)

    out_p = pl.pallas_call(
        _mlp_kernel,
        out_shape=jax.ShapeDtypeStruct((N_pad, O_pad), _F32),
        grid=(n_node_tiles,),
        in_specs=[
            pl.BlockSpec((TN, H_pad), lambda i: (i, 0)),
            pl.BlockSpec((TN, H_pad), lambda i: (i, 0)),
            pl.BlockSpec((TN, H_pad), lambda i: (i, 0)),
            pl.BlockSpec((1, H_pad), lambda i: (0, 0)),
            pl.BlockSpec((H_pad, H2_pad), lambda i: (0, 0)),
            pl.BlockSpec((1, H2_pad), lambda i: (0, 0)),
            pl.BlockSpec((H2_pad, O_pad), lambda i: (0, 0)),
            pl.BlockSpec((1, O_pad), lambda i: (0, 0)),
        ],
        out_specs=pl.BlockSpec((TN, O_pad), lambda i: (i, 0)),
        compiler_params=pltpu.CompilerParams(
            dimension_semantics=("parallel",), vmem_limit_bytes=32 << 20),
    )(h, agg_a, agg_b, ope_row, w1f_p, b1f_p, w23_p, b23_p)

    return out_p[:N, :out_dim]


# speedup vs baseline: 9.2817x; 1.1639x over previous
import functools

import jax
import jax.numpy as jnp
from jax import lax
from jax.experimental import pallas as pl
from jax.experimental.pallas import tpu as pltpu

_BF16 = jnp.bfloat16
_F32 = jnp.float32


def _round_up(x, m):
    return (x + m - 1) // m * m


def _encode_kernel(x_ref, midx_ref, alpha_ref, w_enc_ref, dec_tok_ref, h_ref,
                   *, tn):
    x = x_ref[...]
    a = jnp.where(x >= 0.0, x, alpha_ref[...] * x)
    h = jnp.dot(a.astype(_BF16), w_enc_ref[...], preferred_element_type=_F32)
    row_ids = (lax.broadcasted_iota(jnp.int32, (tn, midx_ref.shape[1]), 0)
               + pl.program_id(0) * tn)
    is_masked = jnp.max((row_ids == midx_ref[...]).astype(_F32), axis=1,
                        keepdims=True)
    h_ref[...] = jnp.where(is_masked > 0.0, dec_tok_ref[...], h)


def _message_kernel(ei_ref, c_ref, h_ref, tab_ref, agg0_ref, agg1_ref, *,
                    n_edges, unroll):
    agg0_ref[...] = jnp.zeros(agg0_ref.shape, agg0_ref.dtype)
    agg1_ref[...] = jnp.zeros(agg1_ref.shape, agg1_ref.dtype)
    aggs = (agg0_ref, agg1_ref)

    def body(i, carry):
        base = i * unroll
        for j in range(unroll):
            s = ei_ref[base + j]
            d = ei_ref[n_edges + base + j]
            cc = c_ref[base + j]
            row = h_ref[pl.ds(s, 1), :, :] + tab_ref[pl.ds(cc, 1), :, :]
            m = jnp.maximum(row, 0.0)
            a = aggs[j % 2]
            a[pl.ds(d, 1), :, :] = a[pl.ds(d, 1), :, :] + m
        return carry

    lax.fori_loop(0, n_edges // unroll, body, 0)


def _mlp_kernel(h_ref, agg0_ref, agg1_ref, ope_ref, w1_ref, b1_ref, w23_ref,
                b23_ref, out_ref):
    pre = ope_ref[...] * h_ref[...] + (agg0_ref[...] + agg1_ref[...])
    hid = jnp.maximum(
        jnp.dot(pre.astype(_BF16), w1_ref[...], preferred_element_type=_F32)
        + b1_ref[...], 0.0)
    out_ref[...] = (jnp.dot(hid.astype(_BF16), w23_ref[...],
                            preferred_element_type=_F32) + b23_ref[...])


def kernel(alpha, eps, dec_token, w_enc, w1, b1, bn_scale, bn_shift, w2, b2,
           w_out, b_out, bond_emb_0, bond_emb_1, bond_emb_2,
           x, edge_index, edge_attr, masked_node_indices):
    N, H = x.shape
    E = edge_index.shape[1]
    out_dim = w_out.shape[1]
    n_masked = masked_node_indices.shape[0]

    H_pad = _round_up(H, 128)
    H2_pad = _round_up(2 * H, 128)
    O_pad = _round_up(out_dim, 128)
    M_pad = _round_up(n_masked, 128)

    TN = 512
    UNROLL = 64

    N_pad = _round_up(N, TN)
    E_pad = _round_up(E, UNROLL)

    def pad2(a, r, c):
        a = a.astype(_F32)
        if a.shape == (r, c):
            return a
        return jnp.zeros((r, c), _F32).at[:a.shape[0], :a.shape[1]].set(a)

    x_p = pad2(x, N_pad, H_pad)
    midx = jnp.full((1, M_pad), -1, jnp.int32).at[0, :n_masked].set(
        masked_node_indices.astype(jnp.int32))
    w_enc_p = pad2(w_enc, H_pad, H_pad).astype(_BF16)
    dec_tok_p = pad2(dec_token, 1, H_pad)

    w1f = w1 * bn_scale
    b1f = b1 * bn_scale + bn_shift
    w23 = w2 @ w_out
    b23 = b2 @ w_out + b_out
    w1f_p = pad2(w1f, H_pad, H2_pad).astype(_BF16)
    b1f_p = pad2(b1f, 1, H2_pad)
    w23_p = pad2(w23, H2_pad, O_pad).astype(_BF16)
    b23_p = pad2(b23, 1, O_pad)

    v0 = bond_emb_0.shape[0]
    v1 = bond_emb_1.shape[0]
    v2 = bond_emb_2.shape[0]
    nv = v0 * v1 * v2
    nv_pad = _round_up(nv + 1, 8)
    tabc = (bond_emb_0.astype(_F32)[:, None, None, :]
            + bond_emb_1.astype(_F32)[None, :, None, :]
            + bond_emb_2.astype(_F32)[None, None, :, :]).reshape(nv, H)
    tabc_p = pad2(tabc, nv_pad, H_pad).reshape(nv_pad, 1, H_pad)

    ea = edge_attr.astype(jnp.int32)
    c_ids = (ea[:, 0] * (v1 * v2) + ea[:, 1] * v2 + ea[:, 2]).astype(jnp.int32)
    n_rows = N_pad + 8 if E_pad > E else N_pad
    if E_pad == E:
        ei_flat = edge_index.astype(jnp.int32).reshape(2 * E)
        c_p = c_ids
    else:
        ei_flat = (jnp.full((2, E_pad), jnp.int32(N_pad), jnp.int32)
                   .at[:, :E].set(edge_index.astype(jnp.int32))
                   .at[0, E:].set(0).reshape(2 * E_pad))
        c_p = jnp.full((E_pad,), nv, jnp.int32).at[:E].set(c_ids)

    alpha_row = jnp.broadcast_to(alpha.astype(_F32), (1, H_pad))
    ope_row = jnp.broadcast_to((1.0 + eps).astype(_F32), (1, H_pad))

    n_node_tiles = N_pad // TN

    h = pl.pallas_call(
        functools.partial(_encode_kernel, tn=TN),
        out_shape=jax.ShapeDtypeStruct((N_pad, H_pad), _F32),
        grid=(n_node_tiles,),
        in_specs=[
            pl.BlockSpec((TN, H_pad), lambda i: (i, 0)),
            pl.BlockSpec((1, M_pad), lambda i: (0, 0)),
            pl.BlockSpec((1, H_pad), lambda i: (0, 0)),
            pl.BlockSpec((H_pad, H_pad), lambda i: (0, 0)),
            pl.BlockSpec((1, H_pad), lambda i: (0, 0)),
        ],
        out_specs=pl.BlockSpec((TN, H_pad), lambda i: (i, 0)),
        compiler_params=pltpu.CompilerParams(
            dimension_semantics=("parallel",), vmem_limit_bytes=32 << 20),
    )(x_p, midx, alpha_row, w_enc_p, dec_tok_p)

    h3 = h.reshape(N_pad, 1, H_pad)
    agg3a, agg3b = pl.pallas_call(
        functools.partial(_message_kernel, n_edges=E_pad, unroll=UNROLL),
        out_shape=(jax.ShapeDtypeStruct((n_rows, 1, H_pad), _F32),
                   jax.ShapeDtypeStruct((n_rows, 1, H_pad), _F32)),
        grid=(1,),
        in_specs=[
            pl.BlockSpec(memory_space=pltpu.SMEM),
            pl.BlockSpec(memory_space=pltpu.SMEM),
            pl.BlockSpec((N_pad, 1, H_pad), lambda i: (0, 0, 0)),
            pl.BlockSpec((nv_pad, 1, H_pad), lambda i: (0, 0, 0)),
        ],
        out_specs=(pl.BlockSpec((n_rows, 1, H_pad), lambda i: (0, 0, 0)),
                   pl.BlockSpec((n_rows, 1, H_pad), lambda i: (0, 0, 0))),
        compiler_params=pltpu.CompilerParams(
            dimension_semantics=("arbitrary",), vmem_limit_bytes=56 << 20),
    )(ei_flat, c_p, h3, tabc_p)

    def flat(a):
        a = a[:N_pad] if n_rows != N_pad else a
        return a.reshape(N_pad, H_pad)

    agg_a, agg_b = flat(agg3a), flat(agg3b)

    out_p = pl.pallas_call(
        _mlp_kernel,
        out_shape=jax.ShapeDtypeStruct((N_pad, O_pad), _F32),
        grid=(n_node_tiles,),
        in_specs=[
            pl.BlockSpec((TN, H_pad), lambda i: (i, 0)),
            pl.BlockSpec((TN, H_pad), lambda i: (i, 0)),
            pl.BlockSpec((TN, H_pad), lambda i: (i, 0)),
            pl.BlockSpec((1, H_pad), lambda i: (0, 0)),
            pl.BlockSpec((H_pad, H2_pad), lambda i: (0, 0)),
            pl.BlockSpec((1, H2_pad), lambda i: (0, 0)),
            pl.BlockSpec((H2_pad, O_pad), lambda i: (0, 0)),
            pl.BlockSpec((1, O_pad), lambda i: (0, 0)),
        ],
        out_specs=pl.BlockSpec((TN, O_pad), lambda i: (i, 0)),
        compiler_params=pltpu.CompilerParams(
            dimension_semantics=("parallel",), vmem_limit_bytes=32 << 20),
    )(h, agg_a, agg_b, ope_row, w1f_p, b1f_p, w23_p, b23_p)

    return out_p[:N, :out_dim]


# speedup vs baseline: 9.7924x; 1.0550x over previous
import functools

import jax
import jax.numpy as jnp
from jax import lax
from jax.experimental import pallas as pl
from jax.experimental.pallas import tpu as pltpu

_BF16 = jnp.bfloat16
_F32 = jnp.float32


def _round_up(x, m):
    return (x + m - 1) // m * m


def _encode_kernel(x_ref, midx_ref, alpha_ref, w_enc_ref, dec_tok_ref, h_ref,
                   *, tn):
    x = x_ref[...]
    a = jnp.where(x >= 0.0, x, alpha_ref[...] * x)
    h = jnp.dot(a.astype(_BF16), w_enc_ref[...], preferred_element_type=_F32)
    row_ids = (lax.broadcasted_iota(jnp.int32, (tn, midx_ref.shape[1]), 0)
               + pl.program_id(0) * tn)
    is_masked = jnp.max((row_ids == midx_ref[...]).astype(_F32), axis=1,
                        keepdims=True)
    h_ref[...] = jnp.where(is_masked > 0.0, dec_tok_ref[...], h)


def _message_kernel(ei_ref, c_ref, h_ref, tab_ref, agg0_ref, agg1_ref, *,
                    n_edges, unroll):
    agg0_ref[...] = jnp.zeros(agg0_ref.shape, agg0_ref.dtype)
    agg1_ref[...] = jnp.zeros(agg1_ref.shape, agg1_ref.dtype)
    aggs = (agg0_ref, agg1_ref)

    def body(i, carry):
        base = i * unroll
        for j in range(unroll):
            s = ei_ref[base + j]
            d = ei_ref[n_edges + base + j]
            cc = c_ref[base + j]
            row = h_ref[pl.ds(s, 1), :, :] + tab_ref[pl.ds(cc, 1), :, :]
            m = jnp.maximum(row, 0.0)
            a = aggs[j % 2]
            a[pl.ds(d, 1), :, :] = a[pl.ds(d, 1), :, :] + m
        return carry

    lax.fori_loop(0, n_edges // unroll, body, 0)


def _mlp_kernel(h_ref, agg0_ref, agg1_ref, ope_ref, w1_ref, b1_ref, w23_ref,
                b23_ref, out_ref):
    pre = ope_ref[...] * h_ref[...] + (agg0_ref[...] + agg1_ref[...])
    hid = jnp.maximum(
        jnp.dot(pre.astype(_BF16), w1_ref[...], preferred_element_type=_F32)
        + b1_ref[...], 0.0)
    out_ref[...] = (jnp.dot(hid.astype(_BF16), w23_ref[...],
                            preferred_element_type=_F32) + b23_ref[...])


def kernel(alpha, eps, dec_token, w_enc, w1, b1, bn_scale, bn_shift, w2, b2,
           w_out, b_out, bond_emb_0, bond_emb_1, bond_emb_2,
           x, edge_index, edge_attr, masked_node_indices):
    N, H = x.shape
    E = edge_index.shape[1]
    out_dim = w_out.shape[1]
    n_masked = masked_node_indices.shape[0]

    H_pad = _round_up(H, 128)
    H2_pad = _round_up(2 * H, 128)
    O_pad = _round_up(out_dim, 128)
    M_pad = _round_up(n_masked, 128)

    TN = 1024
    UNROLL = 64

    N_pad = _round_up(N, TN)
    E_pad = _round_up(E, UNROLL)

    def pad2(a, r, c):
        a = a.astype(_F32)
        if a.shape == (r, c):
            return a
        return jnp.zeros((r, c), _F32).at[:a.shape[0], :a.shape[1]].set(a)

    x_p = pad2(x, N_pad, H_pad)
    midx = jnp.full((1, M_pad), -1, jnp.int32).at[0, :n_masked].set(
        masked_node_indices.astype(jnp.int32))
    w_enc_p = pad2(w_enc, H_pad, H_pad).astype(_BF16)
    dec_tok_p = pad2(dec_token, 1, H_pad)

    w1f = w1 * bn_scale
    b1f = b1 * bn_scale + bn_shift
    w23 = w2 @ w_out
    b23 = b2 @ w_out + b_out
    w1f_p = pad2(w1f, H_pad, H2_pad).astype(_BF16)
    b1f_p = pad2(b1f, 1, H2_pad)
    w23_p = pad2(w23, H2_pad, O_pad).astype(_BF16)
    b23_p = pad2(b23, 1, O_pad)

    v0 = bond_emb_0.shape[0]
    v1 = bond_emb_1.shape[0]
    v2 = bond_emb_2.shape[0]
    nv = v0 * v1 * v2
    nv_pad = _round_up(nv + 1, 8)
    tabc = (bond_emb_0.astype(_F32)[:, None, None, :]
            + bond_emb_1.astype(_F32)[None, :, None, :]
            + bond_emb_2.astype(_F32)[None, None, :, :]).reshape(nv, H)
    tabc_p = pad2(tabc, nv_pad, H_pad).reshape(nv_pad, 1, H_pad)

    ea = edge_attr.astype(jnp.int32)
    c_ids = (ea[:, 0] * (v1 * v2) + ea[:, 1] * v2 + ea[:, 2]).astype(jnp.int32)
    n_rows = N_pad + 8 if E_pad > E else N_pad
    if E_pad == E:
        ei_flat = edge_index.astype(jnp.int32).reshape(2 * E)
        c_p = c_ids
    else:
        ei_flat = (jnp.full((2, E_pad), jnp.int32(N_pad), jnp.int32)
                   .at[:, :E].set(edge_index.astype(jnp.int32))
                   .at[0, E:].set(0).reshape(2 * E_pad))
        c_p = jnp.full((E_pad,), nv, jnp.int32).at[:E].set(c_ids)

    alpha_row = jnp.broadcast_to(alpha.astype(_F32), (1, H_pad))
    ope_row = jnp.broadcast_to((1.0 + eps).astype(_F32), (1, H_pad))

    n_node_tiles = N_pad // TN

    h = pl.pallas_call(
        functools.partial(_encode_kernel, tn=TN),
        out_shape=jax.ShapeDtypeStruct((N_pad, H_pad), _F32),
        grid=(n_node_tiles,),
        in_specs=[
            pl.BlockSpec((TN, H_pad), lambda i: (i, 0)),
            pl.BlockSpec((1, M_pad), lambda i: (0, 0)),
            pl.BlockSpec((1, H_pad), lambda i: (0, 0)),
            pl.BlockSpec((H_pad, H_pad), lambda i: (0, 0)),
            pl.BlockSpec((1, H_pad), lambda i: (0, 0)),
        ],
        out_specs=pl.BlockSpec((TN, H_pad), lambda i: (i, 0)),
        compiler_params=pltpu.CompilerParams(
            dimension_semantics=("parallel",), vmem_limit_bytes=32 << 20),
    )(x_p, midx, alpha_row, w_enc_p, dec_tok_p)

    h3 = h.reshape(N_pad, 1, H_pad)
    agg3a, agg3b = pl.pallas_call(
        functools.partial(_message_kernel, n_edges=E_pad, unroll=UNROLL),
        out_shape=(jax.ShapeDtypeStruct((n_rows, 1, H_pad), _F32),
                   jax.ShapeDtypeStruct((n_rows, 1, H_pad), _F32)),
        grid=(1,),
        in_specs=[
            pl.BlockSpec(memory_space=pltpu.SMEM),
            pl.BlockSpec(memory_space=pltpu.SMEM),
            pl.BlockSpec((N_pad, 1, H_pad), lambda i: (0, 0, 0)),
            pl.BlockSpec((nv_pad, 1, H_pad), lambda i: (0, 0, 0)),
        ],
        out_specs=(pl.BlockSpec((n_rows, 1, H_pad), lambda i: (0, 0, 0)),
                   pl.BlockSpec((n_rows, 1, H_pad), lambda i: (0, 0, 0))),
        compiler_params=pltpu.CompilerParams(
            dimension_semantics=("arbitrary",), vmem_limit_bytes=56 << 20),
    )(ei_flat, c_p, h3, tabc_p)

    def flat(a):
        a = a[:N_pad] if n_rows != N_pad else a
        return a.reshape(N_pad, H_pad)

    agg_a, agg_b = flat(agg3a), flat(agg3b)

    out_p = pl.pallas_call(
        _mlp_kernel,
        out_shape=jax.ShapeDtypeStruct((N_pad, O_pad), _F32),
        grid=(n_node_tiles,),
        in_specs=[
            pl.BlockSpec((TN, H_pad), lambda i: (i, 0)),
            pl.BlockSpec((TN, H_pad), lambda i: (i, 0)),
            pl.BlockSpec((TN, H_pad), lambda i: (i, 0)),
            pl.BlockSpec((1, H_pad), lambda i: (0, 0)),
            pl.BlockSpec((H_pad, H2_pad), lambda i: (0, 0)),
            pl.BlockSpec((1, H2_pad), lambda i: (0, 0)),
            pl.BlockSpec((H2_pad, O_pad), lambda i: (0, 0)),
            pl.BlockSpec((1, O_pad), lambda i: (0, 0)),
        ],
        out_specs=pl.BlockSpec((TN, O_pad), lambda i: (i, 0)),
        compiler_params=pltpu.CompilerParams(
            dimension_semantics=("parallel",), vmem_limit_bytes=32 << 20),
    )(h, agg_a, agg_b, ope_row, w1f_p, b1f_p, w23_p, b23_p)

    return out_p[:N, :out_dim]


# speedup vs baseline: 10.2768x; 1.0495x over previous
import functools

import jax
import jax.numpy as jnp
from jax import lax
from jax.experimental import pallas as pl
from jax.experimental.pallas import tpu as pltpu

_BF16 = jnp.bfloat16
_F32 = jnp.float32


def _round_up(x, m):
    return (x + m - 1) // m * m


def _encode_kernel(x_ref, midx_ref, alpha_ref, w_enc_ref, dec_tok_ref, h_ref,
                   *, tn):
    x = x_ref[...]
    a = jnp.where(x >= 0.0, x, alpha_ref[...] * x)
    h = jnp.dot(a.astype(_BF16), w_enc_ref[...], preferred_element_type=_F32)
    row_ids = (lax.broadcasted_iota(jnp.int32, (tn, midx_ref.shape[1]), 0)
               + pl.program_id(0) * tn)
    is_masked = jnp.max((row_ids == midx_ref[...]).astype(_F32), axis=1,
                        keepdims=True)
    h_ref[...] = jnp.where(is_masked > 0.0, dec_tok_ref[...], h)


def _message_kernel(ei_ref, c_ref, h_ref, tab_ref, agg0_ref, agg1_ref, *,
                    n_edges, unroll):
    agg0_ref[...] = jnp.zeros(agg0_ref.shape, agg0_ref.dtype)
    agg1_ref[...] = jnp.zeros(agg1_ref.shape, agg1_ref.dtype)
    aggs = (agg0_ref, agg1_ref)

    def body(i, carry):
        base = i * unroll
        for j in range(unroll):
            s = ei_ref[base + j]
            d = ei_ref[n_edges + base + j]
            cc = c_ref[base + j]
            row = h_ref[pl.ds(s, 1), :, :] + tab_ref[pl.ds(cc, 1), :, :]
            m = jnp.maximum(row, 0.0)
            a = aggs[j % 2]
            a[pl.ds(d, 1), :, :] = a[pl.ds(d, 1), :, :] + m
        return carry

    lax.fori_loop(0, n_edges // unroll, body, 0)


def _mlp_kernel(h_ref, agg0_ref, agg1_ref, ope_ref, w1_ref, b1_ref, w23_ref,
                b23_ref, out_ref):
    pre = ope_ref[...] * h_ref[...] + (agg0_ref[...] + agg1_ref[...])
    hid = jnp.maximum(
        jnp.dot(pre.astype(_BF16), w1_ref[...], preferred_element_type=_F32)
        + b1_ref[...], 0.0)
    out_ref[...] = (jnp.dot(hid.astype(_BF16), w23_ref[...],
                            preferred_element_type=_F32) + b23_ref[...])


def kernel(alpha, eps, dec_token, w_enc, w1, b1, bn_scale, bn_shift, w2, b2,
           w_out, b_out, bond_emb_0, bond_emb_1, bond_emb_2,
           x, edge_index, edge_attr, masked_node_indices):
    N, H = x.shape
    E = edge_index.shape[1]
    out_dim = w_out.shape[1]
    n_masked = masked_node_indices.shape[0]

    H_pad = _round_up(H, 128)
    H2_pad = _round_up(2 * H, 128)
    O_pad = _round_up(out_dim, 128)
    M_pad = _round_up(n_masked, 128)

    TN = 4096
    UNROLL = 64

    N_pad = _round_up(N, TN)
    E_pad = _round_up(E, UNROLL)

    def pad2(a, r, c):
        a = a.astype(_F32)
        if a.shape == (r, c):
            return a
        return jnp.zeros((r, c), _F32).at[:a.shape[0], :a.shape[1]].set(a)

    x_p = pad2(x, N_pad, H_pad)
    midx = jnp.full((1, M_pad), -1, jnp.int32).at[0, :n_masked].set(
        masked_node_indices.astype(jnp.int32))
    w_enc_p = pad2(w_enc, H_pad, H_pad).astype(_BF16)
    dec_tok_p = pad2(dec_token, 1, H_pad)

    w1f = w1 * bn_scale
    b1f = b1 * bn_scale + bn_shift
    w23 = w2 @ w_out
    b23 = b2 @ w_out + b_out
    w1f_p = pad2(w1f, H_pad, H2_pad).astype(_BF16)
    b1f_p = pad2(b1f, 1, H2_pad)
    w23_p = pad2(w23, H2_pad, O_pad).astype(_BF16)
    b23_p = pad2(b23, 1, O_pad)

    v0 = bond_emb_0.shape[0]
    v1 = bond_emb_1.shape[0]
    v2 = bond_emb_2.shape[0]
    nv = v0 * v1 * v2
    nv_pad = _round_up(nv + 1, 8)
    tabc = (bond_emb_0.astype(_F32)[:, None, None, :]
            + bond_emb_1.astype(_F32)[None, :, None, :]
            + bond_emb_2.astype(_F32)[None, None, :, :]).reshape(nv, H)
    tabc_p = pad2(tabc, nv_pad, H_pad).reshape(nv_pad, 1, H_pad)

    ea = edge_attr.astype(jnp.int32)
    c_ids = (ea[:, 0] * (v1 * v2) + ea[:, 1] * v2 + ea[:, 2]).astype(jnp.int32)
    n_rows = N_pad + 8 if E_pad > E else N_pad
    if E_pad == E:
        ei_flat = edge_index.astype(jnp.int32).reshape(2 * E)
        c_p = c_ids
    else:
        ei_flat = (jnp.full((2, E_pad), jnp.int32(N_pad), jnp.int32)
                   .at[:, :E].set(edge_index.astype(jnp.int32))
                   .at[0, E:].set(0).reshape(2 * E_pad))
        c_p = jnp.full((E_pad,), nv, jnp.int32).at[:E].set(c_ids)

    alpha_row = jnp.broadcast_to(alpha.astype(_F32), (1, H_pad))
    ope_row = jnp.broadcast_to((1.0 + eps).astype(_F32), (1, H_pad))

    n_node_tiles = N_pad // TN

    h = pl.pallas_call(
        functools.partial(_encode_kernel, tn=TN),
        out_shape=jax.ShapeDtypeStruct((N_pad, H_pad), _F32),
        grid=(n_node_tiles,),
        in_specs=[
            pl.BlockSpec((TN, H_pad), lambda i: (i, 0)),
            pl.BlockSpec((1, M_pad), lambda i: (0, 0)),
            pl.BlockSpec((1, H_pad), lambda i: (0, 0)),
            pl.BlockSpec((H_pad, H_pad), lambda i: (0, 0)),
            pl.BlockSpec((1, H_pad), lambda i: (0, 0)),
        ],
        out_specs=pl.BlockSpec((TN, H_pad), lambda i: (i, 0)),
        compiler_params=pltpu.CompilerParams(
            dimension_semantics=("parallel",), vmem_limit_bytes=48 << 20),
    )(x_p, midx, alpha_row, w_enc_p, dec_tok_p)

    h3 = h.reshape(N_pad, 1, H_pad)
    agg3a, agg3b = pl.pallas_call(
        functools.partial(_message_kernel, n_edges=E_pad, unroll=UNROLL),
        out_shape=(jax.ShapeDtypeStruct((n_rows, 1, H_pad), _F32),
                   jax.ShapeDtypeStruct((n_rows, 1, H_pad), _F32)),
        grid=(1,),
        in_specs=[
            pl.BlockSpec(memory_space=pltpu.SMEM),
            pl.BlockSpec(memory_space=pltpu.SMEM),
            pl.BlockSpec((N_pad, 1, H_pad), lambda i: (0, 0, 0)),
            pl.BlockSpec((nv_pad, 1, H_pad), lambda i: (0, 0, 0)),
        ],
        out_specs=(pl.BlockSpec((n_rows, 1, H_pad), lambda i: (0, 0, 0)),
                   pl.BlockSpec((n_rows, 1, H_pad), lambda i: (0, 0, 0))),
        compiler_params=pltpu.CompilerParams(
            dimension_semantics=("arbitrary",), vmem_limit_bytes=56 << 20),
    )(ei_flat, c_p, h3, tabc_p)

    def flat(a):
        a = a[:N_pad] if n_rows != N_pad else a
        return a.reshape(N_pad, H_pad)

    agg_a, agg_b = flat(agg3a), flat(agg3b)

    out_p = pl.pallas_call(
        _mlp_kernel,
        out_shape=jax.ShapeDtypeStruct((N_pad, O_pad), _F32),
        grid=(n_node_tiles,),
        in_specs=[
            pl.BlockSpec((TN, H_pad), lambda i: (i, 0)),
            pl.BlockSpec((TN, H_pad), lambda i: (i, 0)),
            pl.BlockSpec((TN, H_pad), lambda i: (i, 0)),
            pl.BlockSpec((1, H_pad), lambda i: (0, 0)),
            pl.BlockSpec((H_pad, H2_pad), lambda i: (0, 0)),
            pl.BlockSpec((1, H2_pad), lambda i: (0, 0)),
            pl.BlockSpec((H2_pad, O_pad), lambda i: (0, 0)),
            pl.BlockSpec((1, O_pad), lambda i: (0, 0)),
        ],
        out_specs=pl.BlockSpec((TN, O_pad), lambda i: (i, 0)),
        compiler_params=pltpu.CompilerParams(
            dimension_semantics=("parallel",), vmem_limit_bytes=48 << 20),
    )(h, agg_a, agg_b, ope_row, w1f_p, b1f_p, w23_p, b23_p)

    return out_p[:N, :out_dim]


# speedup vs baseline: 10.4026x; 1.0122x over previous
import functools

import jax
import jax.numpy as jnp
from jax import lax
from jax.experimental import pallas as pl
from jax.experimental.pallas import tpu as pltpu

_BF16 = jnp.bfloat16
_F32 = jnp.float32


def _round_up(x, m):
    return (x + m - 1) // m * m


def _encode_kernel(x_ref, midx_ref, alpha_ref, w_enc_ref, dec_tok_ref, h_ref,
                   *, tn):
    x = x_ref[...]
    a = jnp.where(x >= 0.0, x, alpha_ref[...] * x)
    h = jnp.dot(a.astype(_BF16), w_enc_ref[...], preferred_element_type=_F32)
    row_ids = (lax.broadcasted_iota(jnp.int32, (tn, midx_ref.shape[1]), 0)
               + pl.program_id(0) * tn)
    is_masked = jnp.max((row_ids == midx_ref[...]).astype(_F32), axis=1,
                        keepdims=True)
    h_ref[...] = jnp.where(is_masked > 0.0, dec_tok_ref[...], h)


def _message_kernel(ei_ref, c_ref, h_ref, tab_ref, agg0_ref, agg1_ref, *,
                    n_edges, unroll):
    agg0_ref[...] = jnp.zeros(agg0_ref.shape, agg0_ref.dtype)
    agg1_ref[...] = jnp.zeros(agg1_ref.shape, agg1_ref.dtype)
    aggs = (agg0_ref, agg1_ref)

    def body(i, carry):
        base = i * unroll
        for j in range(unroll):
            s = ei_ref[base + j]
            d = ei_ref[n_edges + base + j]
            cc = c_ref[base + j]
            row = h_ref[pl.ds(s, 1), :, :] + tab_ref[pl.ds(cc, 1), :, :]
            m = jnp.maximum(row, 0.0)
            a = aggs[j % 2]
            a[pl.ds(d, 1), :, :] = a[pl.ds(d, 1), :, :] + m
        return carry

    lax.fori_loop(0, n_edges // unroll, body, 0)


def _mlp_kernel(h_ref, agg0_ref, agg1_ref, ope_ref, w1_ref, b1_ref, w23_ref,
                b23_ref, out_ref):
    pre = ope_ref[...] * h_ref[...] + (agg0_ref[...] + agg1_ref[...])
    hid = jnp.maximum(
        jnp.dot(pre.astype(_BF16), w1_ref[...], preferred_element_type=_F32)
        + b1_ref[...], 0.0)
    out_ref[...] = (jnp.dot(hid.astype(_BF16), w23_ref[...],
                            preferred_element_type=_F32) + b23_ref[...])


def kernel(alpha, eps, dec_token, w_enc, w1, b1, bn_scale, bn_shift, w2, b2,
           w_out, b_out, bond_emb_0, bond_emb_1, bond_emb_2,
           x, edge_index, edge_attr, masked_node_indices):
    N, H = x.shape
    E = edge_index.shape[1]
    out_dim = w_out.shape[1]
    n_masked = masked_node_indices.shape[0]

    H_pad = _round_up(H, 128)
    H2_pad = _round_up(2 * H, 128)
    O_pad = _round_up(out_dim, 128)
    M_pad = _round_up(n_masked, 128)

    TN = 4096
    UNROLL = 128

    N_pad = _round_up(N, TN)
    E_pad = _round_up(E, UNROLL)

    def pad2(a, r, c):
        a = a.astype(_F32)
        if a.shape == (r, c):
            return a
        return jnp.zeros((r, c), _F32).at[:a.shape[0], :a.shape[1]].set(a)

    x_p = pad2(x, N_pad, H_pad)
    midx = jnp.full((1, M_pad), -1, jnp.int32).at[0, :n_masked].set(
        masked_node_indices.astype(jnp.int32))
    w_enc_p = pad2(w_enc, H_pad, H_pad).astype(_BF16)
    dec_tok_p = pad2(dec_token, 1, H_pad)

    w1f = w1 * bn_scale
    b1f = b1 * bn_scale + bn_shift
    w23 = w2 @ w_out
    b23 = b2 @ w_out + b_out
    w1f_p = pad2(w1f, H_pad, H2_pad).astype(_BF16)
    b1f_p = pad2(b1f, 1, H2_pad)
    w23_p = pad2(w23, H2_pad, O_pad).astype(_BF16)
    b23_p = pad2(b23, 1, O_pad)

    v0 = bond_emb_0.shape[0]
    v1 = bond_emb_1.shape[0]
    v2 = bond_emb_2.shape[0]
    nv = v0 * v1 * v2
    nv_pad = _round_up(nv + 1, 8)
    tabc = (bond_emb_0.astype(_F32)[:, None, None, :]
            + bond_emb_1.astype(_F32)[None, :, None, :]
            + bond_emb_2.astype(_F32)[None, None, :, :]).reshape(nv, H)
    tabc_p = pad2(tabc, nv_pad, H_pad).reshape(nv_pad, 1, H_pad)

    ea = edge_attr.astype(jnp.int32)
    c_ids = (ea[:, 0] * (v1 * v2) + ea[:, 1] * v2 + ea[:, 2]).astype(jnp.int32)
    n_rows = N_pad + 8 if E_pad > E else N_pad
    if E_pad == E:
        ei_flat = edge_index.astype(jnp.int32).reshape(2 * E)
        c_p = c_ids
    else:
        ei_flat = (jnp.full((2, E_pad), jnp.int32(N_pad), jnp.int32)
                   .at[:, :E].set(edge_index.astype(jnp.int32))
                   .at[0, E:].set(0).reshape(2 * E_pad))
        c_p = jnp.full((E_pad,), nv, jnp.int32).at[:E].set(c_ids)

    alpha_row = jnp.broadcast_to(alpha.astype(_F32), (1, H_pad))
    ope_row = jnp.broadcast_to((1.0 + eps).astype(_F32), (1, H_pad))

    n_node_tiles = N_pad // TN

    h = pl.pallas_call(
        functools.partial(_encode_kernel, tn=TN),
        out_shape=jax.ShapeDtypeStruct((N_pad, H_pad), _F32),
        grid=(n_node_tiles,),
        in_specs=[
            pl.BlockSpec((TN, H_pad), lambda i: (i, 0)),
            pl.BlockSpec((1, M_pad), lambda i: (0, 0)),
            pl.BlockSpec((1, H_pad), lambda i: (0, 0)),
            pl.BlockSpec((H_pad, H_pad), lambda i: (0, 0)),
            pl.BlockSpec((1, H_pad), lambda i: (0, 0)),
        ],
        out_specs=pl.BlockSpec((TN, H_pad), lambda i: (i, 0)),
        compiler_params=pltpu.CompilerParams(
            dimension_semantics=("parallel",), vmem_limit_bytes=48 << 20),
    )(x_p, midx, alpha_row, w_enc_p, dec_tok_p)

    h3 = h.reshape(N_pad, 1, H_pad)
    agg3a, agg3b = pl.pallas_call(
        functools.partial(_message_kernel, n_edges=E_pad, unroll=UNROLL),
        out_shape=(jax.ShapeDtypeStruct((n_rows, 1, H_pad), _F32),
                   jax.ShapeDtypeStruct((n_rows, 1, H_pad), _F32)),
        grid=(1,),
        in_specs=[
            pl.BlockSpec(memory_space=pltpu.SMEM),
            pl.BlockSpec(memory_space=pltpu.SMEM),
            pl.BlockSpec((N_pad, 1, H_pad), lambda i: (0, 0, 0)),
            pl.BlockSpec((nv_pad, 1, H_pad), lambda i: (0, 0, 0)),
        ],
        out_specs=(pl.BlockSpec((n_rows, 1, H_pad), lambda i: (0, 0, 0)),
                   pl.BlockSpec((n_rows, 1, H_pad), lambda i: (0, 0, 0))),
        compiler_params=pltpu.CompilerParams(
            dimension_semantics=("arbitrary",), vmem_limit_bytes=56 << 20),
    )(ei_flat, c_p, h3, tabc_p)

    def flat(a):
        a = a[:N_pad] if n_rows != N_pad else a
        return a.reshape(N_pad, H_pad)

    agg_a, agg_b = flat(agg3a), flat(agg3b)

    out_p = pl.pallas_call(
        _mlp_kernel,
        out_shape=jax.ShapeDtypeStruct((N_pad, O_pad), _F32),
        grid=(n_node_tiles,),
        in_specs=[
            pl.BlockSpec((TN, H_pad), lambda i: (i, 0)),
            pl.BlockSpec((TN, H_pad), lambda i: (i, 0)),
            pl.BlockSpec((TN, H_pad), lambda i: (i, 0)),
            pl.BlockSpec((1, H_pad), lambda i: (0, 0)),
            pl.BlockSpec((H_pad, H2_pad), lambda i: (0, 0)),
            pl.BlockSpec((1, H2_pad), lambda i: (0, 0)),
            pl.BlockSpec((H2_pad, O_pad), lambda i: (0, 0)),
            pl.BlockSpec((1, O_pad), lambda i: (0, 0)),
        ],
        out_specs=pl.BlockSpec((TN, O_pad), lambda i: (i, 0)),
        compiler_params=pltpu.CompilerParams(
            dimension_semantics=("parallel",), vmem_limit_bytes=48 << 20),
    )(h, agg_a, agg_b, ope_row, w1f_p, b1f_p, w23_p, b23_p)

    return out_p[:N, :out_dim]
